# baseline (speedup 1.0000x reference)
import functools

import jax
import jax.numpy as jnp
from jax import lax
from jax.experimental import pallas as pl
from jax.experimental.pallas import tpu as pltpu
from jax.experimental.pallas import tpu_sc as plsc

NC = 2
NS = 16
L = 16
CH = 128


def _sc_fused(uemb, hemb, ubias, hbias, uidx, hidx):
    B = uidx.shape[0]
    bt = B // NS
    bw = B // (NS * NC)
    nch = bt // CH
    nchw = bw // CH
    mesh = plsc.VectorSubcoreMesh(core_axis_name="c", subcore_axis_name="s")

    @functools.partial(
        pl.kernel,
        out_type=jax.ShapeDtypeStruct((B,), jnp.float32),
        mesh=mesh,
        compiler_params=pltpu.CompilerParams(
            use_tc_tiling_on_sc=False, needs_layout_passes=False),
        scratch_types=[
            pltpu.VMEM((bt,), jnp.int32),
            pltpu.VMEM((bt,), jnp.int32),
            pltpu.VMEM((bt, L), jnp.float32),
            pltpu.VMEM((bt, L), jnp.float32),
            pltpu.VMEM((bw,), jnp.float32),
            pltpu.VMEM((bw,), jnp.float32),
            pltpu.VMEM((L,), jnp.float32),
            pltpu.VMEM((NS, L), jnp.float32),
            pltpu.VMEM((bw,), jnp.float32),
            pltpu.VMEM_SHARED((NS, L), jnp.float32),
            pltpu.SemaphoreType.DMA,
        ],
    )
    def k(uemb_h, hemb_h, ub_h, hb_h, uidx_h, hidx_h, out_h,
          uidx_v, hidx_v, urows, hrows, ubg, hbg, accv, parts, outv,
          shared, sem):
        cid = lax.axis_index("c")
        sid = lax.axis_index("s")
        dot_base = sid * bt
        out_base = sid * bt + cid * bw
        pltpu.sync_copy(uidx_h.at[pl.ds(dot_base, bt)], uidx_v)
        pltpu.sync_copy(hidx_h.at[pl.ds(dot_base, bt)], hidx_v)
        copies = []
        for j in range(nch):
            sl = pl.ds(j * CH, CH)
            copies.append(pltpu.async_copy(uemb_h.at[uidx_v.at[sl]], urows.at[sl], sem))
            copies.append(pltpu.async_copy(hemb_h.at[hidx_v.at[sl]], hrows.at[sl], sem))
        for j in range(nchw):
            sl = pl.ds(cid * bw + j * CH, CH)
            dl = pl.ds(j * CH, CH)
            copies.append(pltpu.async_copy(ub_h.at[uidx_v.at[sl]], ubg.at[dl], sem))
            copies.append(pltpu.async_copy(hb_h.at[hidx_v.at[sl]], hbg.at[dl], sem))
        for c in copies:
            c.wait()

        def dot_body(i, acc):
            return acc + urows[i] * hrows[i]

        accv[...] = lax.fori_loop(0, bt, dot_body, jnp.zeros((L,), jnp.float32))
        pltpu.sync_copy(accv, shared.at[sid])
        plsc.subcore_barrier()
        pltpu.sync_copy(shared, parts)

        def red_body(i, acc):
            return acc + parts[i]

        pvec = lax.fori_loop(0, NS, red_body, jnp.zeros((L,), jnp.float32))
        dn = lax.GatherDimensionNumbers(
            offset_dims=(), collapsed_slice_dims=(0,), start_index_map=(0,))
        s = lax.gather(plsc.cumsum(pvec),
                       jnp.full((L, 1), L - 1, jnp.int32), dn,
                       slice_sizes=(1,),
                       mode=lax.GatherScatterMode.PROMISE_IN_BOUNDS)

        def out_body(i, _):
            x = s + ubg[pl.ds(i * L, L)] + hbg[pl.ds(i * L, L)]
            outv[pl.ds(i * L, L)] = 1.0 / (1.0 + jnp.exp(-x))
            return 0

        lax.fori_loop(0, bw // L, out_body, 0)
        pltpu.sync_copy(outv, out_h.at[pl.ds(out_base, bw)])

    return k(uemb, hemb, ubias, hbias, uidx, hidx)


def kernel(inputs, user_emb, user_bias, hotel_emb, hotel_bias):
    B = inputs.shape[0]
    uidx = inputs[:, 0].astype(jnp.int32)
    hidx = inputs[:, 1].astype(jnp.int32)
    lim = min(user_emb.shape[0], hotel_emb.shape[0])
    out = _sc_fused(
        user_emb[:lim], hotel_emb[:lim],
        user_bias[:lim].reshape(-1), hotel_bias[:lim].reshape(-1),
        uidx, hidx)
    return out.reshape(B, 1)


# speedup vs baseline: 1.2022x; 1.2022x over previous
import functools

import jax
import jax.numpy as jnp
from jax import lax
from jax.experimental import pallas as pl
from jax.experimental.pallas import tpu as pltpu
from jax.experimental.pallas import tpu_sc as plsc

NC = 2
NS = 16
L = 16
E = 16
CH = 128


def _sc_fused(ut_flat, ht_flat, ubias, hbias, uidx, hidx, lim):
    B = uidx.shape[0]
    bt = B // NS
    bw = bt // NC
    nchw = bw // CH
    mesh = plsc.VectorSubcoreMesh(core_axis_name="c", subcore_axis_name="s")

    @functools.partial(
        pl.kernel,
        out_type=jax.ShapeDtypeStruct((B,), jnp.float32),
        mesh=mesh,
        compiler_params=pltpu.CompilerParams(
            use_tc_tiling_on_sc=False, needs_layout_passes=False),
        scratch_types=[
            pltpu.VMEM((bt,), jnp.int32),
            pltpu.VMEM((bt,), jnp.int32),
            pltpu.VMEM((E, bt), jnp.float32),
            pltpu.VMEM((E, bt), jnp.float32),
            pltpu.VMEM((bw,), jnp.float32),
            pltpu.VMEM((bw,), jnp.float32),
            pltpu.VMEM((L,), jnp.float32),
            pltpu.VMEM((NS, L), jnp.float32),
            pltpu.VMEM((bw,), jnp.float32),
            pltpu.VMEM_SHARED((NS, L), jnp.float32),
            pltpu.SemaphoreType.DMA,
        ],
    )
    def k(ut_h, ht_h, ub_h, hb_h, uidx_h, hidx_h, out_h,
          uidx_v, hidx_v, ug, hg, ubg, hbg, accv, parts, outv, shared, sem):
        cid = lax.axis_index("c")
        sid = lax.axis_index("s")
        dot_base = sid * bt
        out_base = sid * bt + cid * bw
        pltpu.sync_copy(uidx_h.at[pl.ds(dot_base, bt)], uidx_v)
        pltpu.sync_copy(hidx_h.at[pl.ds(dot_base, bt)], hidx_v)
        copies = []
        for d in range(E):
            seg = pl.ds(d * lim, lim)
            copies.append(pltpu.async_copy(
                ut_h.at[seg].at[uidx_v], ug.at[d], sem))
            copies.append(pltpu.async_copy(
                ht_h.at[seg].at[hidx_v], hg.at[d], sem))
        for j in range(nchw):
            sl = pl.ds(cid * bw + j * CH, CH)
            dl = pl.ds(j * CH, CH)
            copies.append(pltpu.async_copy(ub_h.at[uidx_v.at[sl]], ubg.at[dl], sem))
            copies.append(pltpu.async_copy(hb_h.at[hidx_v.at[sl]], hbg.at[dl], sem))
        for c in copies:
            c.wait()

        per_d = bt // L

        def dot_body(i, acc):
            d = i // per_d
            o = (i % per_d) * L
            sl = pl.ds(o, L)
            return acc + ug[d, sl] * hg[d, sl]

        accv[...] = lax.fori_loop(0, E * per_d, dot_body,
                                  jnp.zeros((L,), jnp.float32))
        pltpu.sync_copy(accv, shared.at[sid])
        plsc.subcore_barrier()
        pltpu.sync_copy(shared, parts)

        def red_body(i, acc):
            return acc + parts[i]

        pvec = lax.fori_loop(0, NS, red_body, jnp.zeros((L,), jnp.float32))
        dn = lax.GatherDimensionNumbers(
            offset_dims=(), collapsed_slice_dims=(0,), start_index_map=(0,))
        s = lax.gather(plsc.cumsum(pvec),
                       jnp.full((L, 1), L - 1, jnp.int32), dn,
                       slice_sizes=(1,),
                       mode=lax.GatherScatterMode.PROMISE_IN_BOUNDS)

        def out_body(i, _):
            x = s + ubg[pl.ds(i * L, L)] + hbg[pl.ds(i * L, L)]
            outv[pl.ds(i * L, L)] = 1.0 / (1.0 + jnp.exp(-x))
            return 0

        lax.fori_loop(0, bw // L, out_body, 0)
        pltpu.sync_copy(outv, out_h.at[pl.ds(out_base, bw)])

    return k(ut_flat, ht_flat, ubias, hbias, uidx, hidx)


def kernel(inputs, user_emb, user_bias, hotel_emb, hotel_bias):
    B = inputs.shape[0]
    uidx = inputs[:, 0].astype(jnp.int32)
    hidx = inputs[:, 1].astype(jnp.int32)
    lim = min(user_emb.shape[0], hotel_emb.shape[0])
    out = _sc_fused(
        user_emb[:lim].T.reshape(-1), hotel_emb[:lim].T.reshape(-1),
        user_bias[:lim].reshape(-1), hotel_bias[:lim].reshape(-1),
        uidx, hidx, lim)
    return out.reshape(B, 1)


# speedup vs baseline: 1.5279x; 1.2709x over previous
import functools

import jax
import jax.numpy as jnp
from jax import lax
from jax.experimental import pallas as pl
from jax.experimental.pallas import tpu as pltpu
from jax.experimental.pallas import tpu_sc as plsc

NC = 2
NS = 16
NW = NC * NS
L = 16
E = 16
CH = 128


def _sc_gather(ut_flat, ht_flat, ubias, hbias, uidx, hidx, lim):
    B = uidx.shape[0]
    bw = B // NW
    nchw = bw // CH
    mesh = plsc.VectorSubcoreMesh(core_axis_name="c", subcore_axis_name="s")

    @functools.partial(
        pl.kernel,
        out_type=(
            jax.ShapeDtypeStruct((NW * L,), jnp.float32),
            jax.ShapeDtypeStruct((B,), jnp.float32),
            jax.ShapeDtypeStruct((B,), jnp.float32),
        ),
        mesh=mesh,
        compiler_params=pltpu.CompilerParams(
            use_tc_tiling_on_sc=False, needs_layout_passes=False),
        scratch_types=[
            pltpu.VMEM((bw,), jnp.int32),
            pltpu.VMEM((bw,), jnp.int32),
            pltpu.VMEM((E, bw), jnp.float32),
            pltpu.VMEM((E, bw), jnp.float32),
            pltpu.VMEM((bw,), jnp.float32),
            pltpu.VMEM((bw,), jnp.float32),
            pltpu.VMEM((L,), jnp.float32),
            pltpu.SemaphoreType.DMA,
            pltpu.SemaphoreType.DMA,
        ],
    )
    def k(ut_h, ht_h, ub_h, hb_h, uidx_h, hidx_h,
          part_o, ubo, hbo,
          uidx_v, hidx_v, ug, hg, ubg, hbg, accv, sem, bsem):
        cid = lax.axis_index("c")
        sid = lax.axis_index("s")
        wid = sid * NC + cid
        base = wid * bw
        pltpu.sync_copy(uidx_h.at[pl.ds(base, bw)], uidx_v)
        pltpu.sync_copy(hidx_h.at[pl.ds(base, bw)], hidx_v)
        copies = []
        for d in range(E):
            seg = pl.ds(d * lim, lim)
            copies.append(pltpu.async_copy(
                ut_h.at[seg].at[uidx_v], ug.at[d], sem))
            copies.append(pltpu.async_copy(
                ht_h.at[seg].at[hidx_v], hg.at[d], sem))
        bcopies = []
        for j in range(nchw):
            sl = pl.ds(j * CH, CH)
            bcopies.append(pltpu.async_copy(ub_h.at[uidx_v.at[sl]], ubg.at[sl], bsem))
            bcopies.append(pltpu.async_copy(hb_h.at[hidx_v.at[sl]], hbg.at[sl], bsem))

        nvd = bw // L
        acc = jnp.zeros((L,), jnp.float32)
        for d in range(E):
            copies[2 * d].wait()
            copies[2 * d + 1].wait()

            def dot_body(i, a, d=d):
                sl = pl.ds(i * L, L)
                return a + ug[d, sl] * hg[d, sl]

            acc = lax.fori_loop(0, nvd, dot_body, acc)
        accv[...] = acc
        pltpu.sync_copy(accv, part_o.at[pl.ds(wid * L, L)])
        for c in bcopies:
            c.wait()
        pltpu.sync_copy(ubg, ubo.at[pl.ds(base, bw)])
        pltpu.sync_copy(hbg, hbo.at[pl.ds(base, bw)])

    return k(ut_flat, ht_flat, ubias, hbias, uidx, hidx)


def _tc_finish(partials, ub, hb):
    def body(part_ref, ub_ref, hb_ref, o_ref):
        s = jnp.sum(part_ref[...])
        o_ref[...] = jax.nn.sigmoid(ub_ref[...] + hb_ref[...] + s)

    return pl.pallas_call(
        body,
        out_shape=jax.ShapeDtypeStruct(ub.shape, jnp.float32),
    )(partials, ub, hb)


def kernel(inputs, user_emb, user_bias, hotel_emb, hotel_bias):
    B = inputs.shape[0]
    uidx = inputs[:, 0].astype(jnp.int32)
    hidx = inputs[:, 1].astype(jnp.int32)
    lim = min(user_emb.shape[0], hotel_emb.shape[0])
    partials, ubg, hbg = _sc_gather(
        user_emb[:lim].T.reshape(-1), hotel_emb[:lim].T.reshape(-1),
        user_bias[:lim].reshape(-1), hotel_bias[:lim].reshape(-1),
        uidx, hidx, lim)
    out = _tc_finish(partials.reshape(4, 128),
                     ubg.reshape(B // 128, 128),
                     hbg.reshape(B // 128, 128))
    return out.reshape(B, 1)


# speedup vs baseline: 1.6007x; 1.0477x over previous
import functools

import jax
import jax.numpy as jnp
from jax import lax
from jax.experimental import pallas as pl
from jax.experimental.pallas import tpu as pltpu
from jax.experimental.pallas import tpu_sc as plsc

NC = 2
NS = 16
NW = NC * NS
L = 16
E = 16
CH = 128

_MESH = dict(core_axis_name="c", subcore_axis_name="s")
_PARAMS = pltpu.CompilerParams(
    use_tc_tiling_on_sc=False, needs_layout_passes=False)


def _sc_user(ut_flat, ubias, uidx, lim):
    B = uidx.shape[0]
    bw = B // NW

    @functools.partial(
        pl.kernel,
        out_type=(
            jax.ShapeDtypeStruct((B * E,), jnp.float32),
            jax.ShapeDtypeStruct((B,), jnp.float32),
        ),
        mesh=plsc.VectorSubcoreMesh(**_MESH),
        compiler_params=_PARAMS,
        scratch_types=[
            pltpu.VMEM((bw,), jnp.int32),
            pltpu.VMEM((E * bw,), jnp.float32),
            pltpu.VMEM((bw,), jnp.float32),
            pltpu.SemaphoreType.DMA,
        ],
    )
    def k(ut_h, ub_h, uidx_h, ug_o, ubo, uidx_v, ug, ubg, sem):
        wid = lax.axis_index("s") * NC + lax.axis_index("c")
        base = wid * bw
        pltpu.sync_copy(uidx_h.at[pl.ds(base, bw)], uidx_v)
        copies = [pltpu.async_copy(ut_h.at[pl.ds(d * lim, lim)].at[uidx_v],
                                   ug.at[pl.ds(d * bw, bw)], sem)
                  for d in range(E)]
        for j in range(bw // CH):
            sl = pl.ds(j * CH, CH)
            copies.append(pltpu.async_copy(ub_h.at[uidx_v.at[sl]], ubg.at[sl], sem))
        for c in copies:
            c.wait()
        pltpu.sync_copy(ug, ug_o.at[pl.ds(base * E, bw * E)])
        pltpu.sync_copy(ubg, ubo.at[pl.ds(base, bw)])

    return k(ut_flat, ubias, uidx)


def _sc_hotel(ht_flat, hbias, hidx, ug_all, lim):
    B = hidx.shape[0]
    bw = B // NW

    @functools.partial(
        pl.kernel,
        out_type=(
            jax.ShapeDtypeStruct((NW * L,), jnp.float32),
            jax.ShapeDtypeStruct((B,), jnp.float32),
        ),
        mesh=plsc.VectorSubcoreMesh(**_MESH),
        compiler_params=_PARAMS,
        scratch_types=[
            pltpu.VMEM((bw,), jnp.int32),
            pltpu.VMEM((E * bw,), jnp.float32),
            pltpu.VMEM((E * bw,), jnp.float32),
            pltpu.VMEM((bw,), jnp.float32),
            pltpu.VMEM((L,), jnp.float32),
            pltpu.SemaphoreType.DMA,
            pltpu.SemaphoreType.DMA,
        ],
    )
    def k(ht_h, hb_h, hidx_h, ug_h, part_o, hbo,
          hidx_v, hg, ug, hbg, accv, sem, bsem):
        wid = lax.axis_index("s") * NC + lax.axis_index("c")
        base = wid * bw
        pltpu.sync_copy(hidx_h.at[pl.ds(base, bw)], hidx_v)
        ucopy = pltpu.async_copy(ug_h.at[pl.ds(base * E, bw * E)], ug, bsem)
        copies = [pltpu.async_copy(ht_h.at[pl.ds(d * lim, lim)].at[hidx_v],
                                   hg.at[pl.ds(d * bw, bw)], sem)
                  for d in range(E)]
        bcopies = []
        for j in range(bw // CH):
            sl = pl.ds(j * CH, CH)
            bcopies.append(pltpu.async_copy(hb_h.at[hidx_v.at[sl]], hbg.at[sl], bsem))
        ucopy.wait()
        nvd = bw // L
        acc = jnp.zeros((L,), jnp.float32)
        for d in range(E):
            copies[d].wait()

            def dot_body(i, a, d=d):
                sl = pl.ds(d * bw + i * L, L)
                return a + ug[sl] * hg[sl]

            acc = lax.fori_loop(0, nvd, dot_body, acc)
        accv[...] = acc
        pltpu.sync_copy(accv, part_o.at[pl.ds(wid * L, L)])
        for c in bcopies:
            c.wait()
        pltpu.sync_copy(hbg, hbo.at[pl.ds(base, bw)])

    return k(ht_flat, hbias, hidx, ug_all)


def _tc_finish(partials, ub, hb):
    def body(part_ref, ub_ref, hb_ref, o_ref):
        s = jnp.sum(part_ref[...])
        o_ref[...] = jax.nn.sigmoid(ub_ref[...] + hb_ref[...] + s)

    return pl.pallas_call(
        body,
        out_shape=jax.ShapeDtypeStruct(ub.shape, jnp.float32),
    )(partials, ub, hb)


def kernel(inputs, user_emb, user_bias, hotel_emb, hotel_bias):
    B = inputs.shape[0]
    uidx = inputs[:, 0].astype(jnp.int32)
    hidx = inputs[:, 1].astype(jnp.int32)
    lim = min(user_emb.shape[0], hotel_emb.shape[0])
    ug_all, ubg = _sc_user(
        user_emb[:lim].T.reshape(-1), user_bias.reshape(-1)[:lim], uidx, lim)
    partials, hbg = _sc_hotel(
        hotel_emb[:lim].T.reshape(-1), hotel_bias.reshape(-1)[:lim], hidx,
        ug_all, lim)
    out = _tc_finish(partials.reshape(4, 128),
                     ubg.reshape(B // 128, 128),
                     hbg.reshape(B // 128, 128))
    return out.reshape(B, 1)


# speedup vs baseline: 1.6859x; 1.0532x over previous
import functools

import jax
import jax.numpy as jnp
from jax import lax
from jax.experimental import pallas as pl
from jax.experimental.pallas import tpu as pltpu
from jax.experimental.pallas import tpu_sc as plsc

NC = 2
NS = 16
NW = NC * NS
L = 16
E = 16
CH = 128

_MESH = dict(core_axis_name="c", subcore_axis_name="s")
_PARAMS = pltpu.CompilerParams(
    use_tc_tiling_on_sc=False, needs_layout_passes=False)


def _sc_user(ut_flat, ubias, uidx, lim):
    B = uidx.shape[0]
    bw = B // NW

    @functools.partial(
        pl.kernel,
        out_type=(
            jax.ShapeDtypeStruct((B * E,), jnp.float32),
            jax.ShapeDtypeStruct((B,), jnp.float32),
        ),
        mesh=plsc.VectorSubcoreMesh(**_MESH),
        compiler_params=_PARAMS,
        scratch_types=[
            pltpu.VMEM((bw,), jnp.int32),
            pltpu.VMEM((E * bw,), jnp.float32),
            pltpu.VMEM((bw,), jnp.float32),
            pltpu.SemaphoreType.DMA,
        ],
    )
    def k(ut_h, ub_h, uidx_h, ug_o, ubo, uidx_v, ug, ubg, sem):
        wid = lax.axis_index("s") * NC + lax.axis_index("c")
        base = wid * bw
        pltpu.sync_copy(uidx_h.at[pl.ds(base, bw)], uidx_v)
        copies = [pltpu.async_copy(ut_h.at[pl.ds(d * lim, lim)].at[uidx_v],
                                   ug.at[pl.ds(d * bw, bw)], sem)
                  for d in range(E)]
        for j in range(bw // CH):
            sl = pl.ds(j * CH, CH)
            copies.append(pltpu.async_copy(ub_h.at[uidx_v.at[sl]], ubg.at[sl], sem))
        for c in copies:
            c.wait()
        pltpu.sync_copy(ug, ug_o.at[pl.ds(base * E, bw * E)])
        pltpu.sync_copy(ubg, ubo.at[pl.ds(base, bw)])

    return k(ut_flat, ubias, uidx)


def _sc_hotel(ht_flat, hbias, hidx, ug_all, lim):
    B = hidx.shape[0]
    bw = B // NW

    @functools.partial(
        pl.kernel,
        out_type=(
            jax.ShapeDtypeStruct((NW * L,), jnp.float32),
            jax.ShapeDtypeStruct((B,), jnp.float32),
        ),
        mesh=plsc.VectorSubcoreMesh(**_MESH),
        compiler_params=_PARAMS,
        scratch_types=[
            pltpu.VMEM((bw,), jnp.int32),
            pltpu.VMEM((E * bw,), jnp.float32),
            pltpu.VMEM((E * bw,), jnp.float32),
            pltpu.VMEM((bw,), jnp.float32),
            pltpu.VMEM((L,), jnp.float32),
            pltpu.SemaphoreType.DMA,
            pltpu.SemaphoreType.DMA,
        ],
    )
    def k(ht_h, hb_h, hidx_h, ug_h, part_o, hbo,
          hidx_v, hg, ug, hbg, accv, sem, bsem):
        wid = lax.axis_index("s") * NC + lax.axis_index("c")
        base = wid * bw
        pltpu.sync_copy(hidx_h.at[pl.ds(base, bw)], hidx_v)
        ucopy = pltpu.async_copy(ug_h.at[pl.ds(base * E, bw * E)], ug, bsem)
        copies = [pltpu.async_copy(ht_h.at[pl.ds(d * lim, lim)].at[hidx_v],
                                   hg.at[pl.ds(d * bw, bw)], sem)
                  for d in range(E)]
        bcopies = []
        for j in range(bw // CH):
            sl = pl.ds(j * CH, CH)
            bcopies.append(pltpu.async_copy(hb_h.at[hidx_v.at[sl]], hbg.at[sl], bsem))
        ucopy.wait()
        nvd = bw // L
        acc = jnp.zeros((L,), jnp.float32)
        for d in range(E):
            copies[d].wait()

            def dot_body(i, a, d=d):
                sl = pl.ds(d * bw + i * L, L)
                return a + ug[sl] * hg[sl]

            acc = lax.fori_loop(0, nvd, dot_body, acc)
        accv[...] = acc
        pltpu.sync_copy(accv, part_o.at[pl.ds(wid * L, L)])
        for c in bcopies:
            c.wait()
        pltpu.sync_copy(hbg, hbo.at[pl.ds(base, bw)])

    return k(ht_flat, hbias, hidx, ug_all)


def _tc_finish(partials, ub, hb):
    def body(part_ref, ub_ref, hb_ref, o_ref):
        s = jnp.sum(part_ref[...])
        o_ref[...] = jax.nn.sigmoid(ub_ref[...] + hb_ref[...] + s)

    return pl.pallas_call(
        body,
        out_shape=jax.ShapeDtypeStruct(ub.shape, jnp.float32),
    )(partials, ub, hb)


def kernel(inputs, user_emb, user_bias, hotel_emb, hotel_bias):
    B = inputs.shape[0]
    uidx = inputs[:, 0].astype(jnp.int32)
    hidx = inputs[:, 1].astype(jnp.int32)
    lim = min(user_emb.shape[0], hotel_emb.shape[0])
    hg_all, hbg = _sc_user(
        hotel_emb[:lim].T.reshape(-1), hotel_bias.reshape(-1)[:lim], hidx, lim)
    partials, ubg = _sc_hotel(
        user_emb[:lim].T.reshape(-1), user_bias.reshape(-1)[:lim], uidx,
        hg_all, lim)
    out = _tc_finish(partials.reshape(4, 128),
                     ubg.reshape(B // 128, 128),
                     hbg.reshape(B // 128, 128))
    return out.reshape(B, 1)


# speedup vs baseline: 1.7901x; 1.0618x over previous
import functools

import jax
import jax.numpy as jnp
from jax import lax
from jax.experimental import pallas as pl
from jax.experimental.pallas import tpu as pltpu
from jax.experimental.pallas import tpu_sc as plsc

NC = 2
NS = 16
NW = NC * NS
L = 16
E = 16
CH = 128

_MESH = dict(core_axis_name="c", subcore_axis_name="s")
_PARAMS = pltpu.CompilerParams(
    use_tc_tiling_on_sc=False, needs_layout_passes=False)


def _sc_user(ut_flat, ubias, uidx, lim):
    B = uidx.shape[0]
    bw = B // NW

    @functools.partial(
        pl.kernel,
        out_type=(
            jax.ShapeDtypeStruct((B * E,), jnp.float32),
            jax.ShapeDtypeStruct((B,), jnp.float32),
        ),
        mesh=plsc.VectorSubcoreMesh(**_MESH),
        compiler_params=_PARAMS,
        scratch_types=[
            pltpu.VMEM((bw,), jnp.int32),
            pltpu.VMEM((E * bw,), jnp.float32),
            pltpu.VMEM((bw,), jnp.float32),
            pltpu.SemaphoreType.DMA,
        ],
    )
    def k(ut_h, ub_h, uidx_h, ug_o, ubo, uidx_v, ug, ubg, sem):
        wid = lax.axis_index("s") * NC + lax.axis_index("c")
        base = wid * bw
        pltpu.sync_copy(uidx_h.at[pl.ds(base, bw)], uidx_v)
        copies = [pltpu.async_copy(ut_h.at[pl.ds(d * lim, lim)].at[uidx_v],
                                   ug.at[pl.ds(d * bw, bw)], sem)
                  for d in range(E)]
        for j in range(bw // CH):
            sl = pl.ds(j * CH, CH)
            copies.append(pltpu.async_copy(ub_h.at[uidx_v.at[sl]], ubg.at[sl], sem))
        for c in copies:
            c.wait()
        pltpu.sync_copy(ug, ug_o.at[pl.ds(base * E, bw * E)])
        pltpu.sync_copy(ubg, ubo.at[pl.ds(base, bw)])

    return k(ut_flat, ubias, uidx)


def _sc_hotel(ht_flat, hbias, hidx, ug_all, lim):
    B = hidx.shape[0]
    bw = B // NW
    tcols = ht_flat.shape[0] // (2 * 8 * 128)

    @functools.partial(
        pl.kernel,
        out_type=(
            jax.ShapeDtypeStruct((NW * L,), jnp.float32),
            jax.ShapeDtypeStruct((B,), jnp.float32),
        ),
        mesh=plsc.VectorSubcoreMesh(**_MESH),
        compiler_params=_PARAMS,
        scratch_types=[
            pltpu.VMEM((bw,), jnp.int32),
            pltpu.VMEM((bw,), jnp.int32),
            pltpu.VMEM((E * bw,), jnp.float32),
            pltpu.VMEM((E * bw,), jnp.float32),
            pltpu.VMEM((bw,), jnp.float32),
            pltpu.VMEM((L,), jnp.float32),
            pltpu.SemaphoreType.DMA,
            pltpu.SemaphoreType.DMA,
        ],
    )
    def k(ht_h, hb_h, hidx_h, ug_h, part_o, hbo,
          hidx_v, haddr_v, hg, ug, hbg, accv, sem, bsem):
        wid = lax.axis_index("s") * NC + lax.axis_index("c")
        base = wid * bw
        pltpu.sync_copy(hidx_h.at[pl.ds(base, bw)], hidx_v)
        ucopy = pltpu.async_copy(ug_h.at[pl.ds(base * E, bw * E)], ug, bsem)

        def addr_body(i, _):
            sl = pl.ds(i * L, L)
            r = hidx_v[sl]
            haddr_v[sl] = ((r >> 7) << 10) + (r & 127)
            return 0

        lax.fori_loop(0, bw // L, addr_body, 0)
        seg_len = tcols * 8 * 128
        copies = [pltpu.async_copy(
            ht_h.at[pl.ds((d // 8) * seg_len + (d % 8) * 128,
                          seg_len - (d % 8) * 128)].at[haddr_v],
            hg.at[pl.ds(d * bw, bw)], sem)
                  for d in range(E)]
        bcopies = []
        for j in range(bw // CH):
            sl = pl.ds(j * CH, CH)
            bcopies.append(pltpu.async_copy(hb_h.at[hidx_v.at[sl]], hbg.at[sl], bsem))
        ucopy.wait()
        nvd = bw // L
        acc = jnp.zeros((L,), jnp.float32)
        for d in range(E):
            copies[d].wait()

            def dot_body(i, a, d=d):
                sl = pl.ds(d * bw + i * L, L)
                return a + ug[sl] * hg[sl]

            acc = lax.fori_loop(0, nvd, dot_body, acc)
        accv[...] = acc
        pltpu.sync_copy(accv, part_o.at[pl.ds(wid * L, L)])
        for c in bcopies:
            c.wait()
        pltpu.sync_copy(hbg, hbo.at[pl.ds(base, bw)])

    return k(ht_flat, hbias, hidx, ug_all)


def _tc_finish(partials, ub, hb):
    def body(part_ref, ub_ref, hb_ref, o_ref):
        s = jnp.sum(part_ref[...])
        o_ref[...] = jax.nn.sigmoid(ub_ref[...] + hb_ref[...] + s)

    return pl.pallas_call(
        body,
        out_shape=jax.ShapeDtypeStruct(ub.shape, jnp.float32),
    )(partials, ub, hb)


def kernel(inputs, user_emb, user_bias, hotel_emb, hotel_bias):
    B = inputs.shape[0]
    uidx = inputs[:, 0].astype(jnp.int32)
    hidx = inputs[:, 1].astype(jnp.int32)
    lim = min(user_emb.shape[0], hotel_emb.shape[0])
    hg_all, hbg = _sc_user(
        hotel_emb[:lim].T.reshape(-1), hotel_bias.reshape(-1)[:lim], hidx, lim)
    limp = ((lim + 127) // 128) * 128
    u4 = (user_emb[:limp].T.reshape(2, 8, limp // 128, 128)
          .transpose(0, 2, 1, 3).reshape(-1))
    partials, ubg = _sc_hotel(
        u4, user_bias.reshape(-1)[:lim], uidx, hg_all, lim)
    out = _tc_finish(partials.reshape(4, 128),
                     ubg.reshape(B // 128, 128),
                     hbg.reshape(B // 128, 128))
    return out.reshape(B, 1)


# speedup vs baseline: 1.8806x; 1.0506x over previous
import functools

import jax
import jax.numpy as jnp
from jax import lax
from jax.experimental import pallas as pl
from jax.experimental.pallas import tpu as pltpu
from jax.experimental.pallas import tpu_sc as plsc

NC = 2
NS = 16
NW = NC * NS
L = 16
E = 16
CH = 128

_MESH = dict(core_axis_name="c", subcore_axis_name="s")
_PARAMS = pltpu.CompilerParams(
    use_tc_tiling_on_sc=False, needs_layout_passes=False)


def _sc_user(ut_flat, ubias, uidx, lim):
    B = uidx.shape[0]
    bw = B // NW
    tcols = ut_flat.shape[0] // (2 * 8 * 128)

    @functools.partial(
        pl.kernel,
        out_type=(
            jax.ShapeDtypeStruct((B * E,), jnp.float32),
            jax.ShapeDtypeStruct((B,), jnp.float32),
        ),
        mesh=plsc.VectorSubcoreMesh(**_MESH),
        compiler_params=_PARAMS,
        scratch_types=[
            pltpu.VMEM((bw,), jnp.int32),
            pltpu.VMEM((bw,), jnp.int32),
            pltpu.VMEM((E * bw,), jnp.float32),
            pltpu.VMEM((bw,), jnp.float32),
            pltpu.SemaphoreType.DMA,
        ],
    )
    def k(ut_h, ub_h, uidx_h, ug_o, ubo, uidx_v, uaddr_v, ug, ubg, sem):
        wid = lax.axis_index("s") * NC + lax.axis_index("c")
        base = wid * bw
        pltpu.sync_copy(uidx_h.at[pl.ds(base, bw)], uidx_v)

        def addr_body(i, _):
            sl = pl.ds(i * L, L)
            r = uidx_v[sl]
            uaddr_v[sl] = ((r >> 7) << 10) + (r & 127)
            return 0

        lax.fori_loop(0, bw // L, addr_body, 0)
        seg_len = tcols * 8 * 128
        copies = [pltpu.async_copy(
            ut_h.at[pl.ds((d // 8) * seg_len + (d % 8) * 128,
                          seg_len - (d % 8) * 128)].at[uaddr_v],
            ug.at[pl.ds(d * bw, bw)], sem)
                  for d in range(E)]
        for j in range(bw // CH):
            sl = pl.ds(j * CH, CH)
            copies.append(pltpu.async_copy(ub_h.at[uidx_v.at[sl]], ubg.at[sl], sem))
        for c in copies:
            c.wait()
        pltpu.sync_copy(ug, ug_o.at[pl.ds(base * E, bw * E)])
        pltpu.sync_copy(ubg, ubo.at[pl.ds(base, bw)])

    return k(ut_flat, ubias, uidx)


def _sc_hotel(ht_flat, hbias, hidx, ug_all, lim):
    B = hidx.shape[0]
    bw = B // NW
    tcols = ht_flat.shape[0] // (2 * 8 * 128)

    @functools.partial(
        pl.kernel,
        out_type=(
            jax.ShapeDtypeStruct((NW * L,), jnp.float32),
            jax.ShapeDtypeStruct((B,), jnp.float32),
        ),
        mesh=plsc.VectorSubcoreMesh(**_MESH),
        compiler_params=_PARAMS,
        scratch_types=[
            pltpu.VMEM((bw,), jnp.int32),
            pltpu.VMEM((bw,), jnp.int32),
            pltpu.VMEM((E * bw,), jnp.float32),
            pltpu.VMEM((E * bw,), jnp.float32),
            pltpu.VMEM((bw,), jnp.float32),
            pltpu.VMEM((L,), jnp.float32),
            pltpu.SemaphoreType.DMA,
            pltpu.SemaphoreType.DMA,
        ],
    )
    def k(ht_h, hb_h, hidx_h, ug_h, part_o, hbo,
          hidx_v, haddr_v, hg, ug, hbg, accv, sem, bsem):
        wid = lax.axis_index("s") * NC + lax.axis_index("c")
        base = wid * bw
        pltpu.sync_copy(hidx_h.at[pl.ds(base, bw)], hidx_v)
        ucopy = pltpu.async_copy(ug_h.at[pl.ds(base * E, bw * E)], ug, bsem)

        def addr_body(i, _):
            sl = pl.ds(i * L, L)
            r = hidx_v[sl]
            haddr_v[sl] = ((r >> 7) << 10) + (r & 127)
            return 0

        lax.fori_loop(0, bw // L, addr_body, 0)
        seg_len = tcols * 8 * 128
        copies = [pltpu.async_copy(
            ht_h.at[pl.ds((d // 8) * seg_len + (d % 8) * 128,
                          seg_len - (d % 8) * 128)].at[haddr_v],
            hg.at[pl.ds(d * bw, bw)], sem)
                  for d in range(E)]
        bcopies = []
        for j in range(bw // CH):
            sl = pl.ds(j * CH, CH)
            bcopies.append(pltpu.async_copy(hb_h.at[hidx_v.at[sl]], hbg.at[sl], bsem))
        ucopy.wait()
        nvd = bw // L
        acc = jnp.zeros((L,), jnp.float32)
        for d in range(E):
            copies[d].wait()

            def dot_body(i, a, d=d):
                sl = pl.ds(d * bw + i * L, L)
                return a + ug[sl] * hg[sl]

            acc = lax.fori_loop(0, nvd, dot_body, acc)
        accv[...] = acc
        pltpu.sync_copy(accv, part_o.at[pl.ds(wid * L, L)])
        for c in bcopies:
            c.wait()
        pltpu.sync_copy(hbg, hbo.at[pl.ds(base, bw)])

    return k(ht_flat, hbias, hidx, ug_all)


def _tc_finish(partials, ub, hb):
    def body(part_ref, ub_ref, hb_ref, o_ref):
        s = jnp.sum(part_ref[...])
        o_ref[...] = jax.nn.sigmoid(ub_ref[...] + hb_ref[...] + s)

    return pl.pallas_call(
        body,
        out_shape=jax.ShapeDtypeStruct(ub.shape, jnp.float32),
    )(partials, ub, hb)


def kernel(inputs, user_emb, user_bias, hotel_emb, hotel_bias):
    B = inputs.shape[0]
    uidx = inputs[:, 0].astype(jnp.int32)
    hidx = inputs[:, 1].astype(jnp.int32)
    lim = min(user_emb.shape[0], hotel_emb.shape[0])
    limp = ((lim + 127) // 128) * 128

    def tiled_view(t):
        tp = jnp.pad(t, ((0, limp - t.shape[0]), (0, 0))) if t.shape[0] < limp else t[:limp]
        return (tp.T.reshape(2, 8, limp // 128, 128)
                .transpose(0, 2, 1, 3).reshape(-1))

    hg_all, hbg = _sc_user(
        tiled_view(hotel_emb), hotel_bias.reshape(-1)[:lim], hidx, lim)
    partials, ubg = _sc_hotel(
        tiled_view(user_emb), user_bias.reshape(-1)[:lim], uidx, hg_all, lim)
    out = _tc_finish(partials.reshape(4, 128),
                     ubg.reshape(B // 128, 128),
                     hbg.reshape(B // 128, 128))
    return out.reshape(B, 1)
